# baseline (speedup 1.0000x reference)
import functools

import jax
import jax.numpy as jnp
from jax import lax
from jax.experimental import pallas as pl
from jax.experimental.pallas import tpu as pltpu
from jax.experimental.pallas import tpu_sc as plsc

B = 16384
L = 20
N = 20
D = 64
VOCAB = 1000000
WIDE = 128
LANES = 16
NC = 2
NS = 16
NW = NC * NS

ROWS_PER_W = B // NW
CHUNK = 8
NCHUNK = ROWS_PER_W // CHUNK
IDX_PER_CHUNK = CHUNK * L
IDX_PER_W = ROWS_PER_W * L
PART = N * LANES
STREAMS = ((0, 128), (128, 32))

WROWS = 400
WBLOCKS = VOCAB // WROWS
WPW = WBLOCKS // NW
TC_BLOCK = 256


def _widen_sc_body(ctx_hbm, cen_hbm, ctxw_hbm, cenw_hbm, buf, wbuf, sem):
    wid = lax.axis_index("s") * NC + lax.axis_index("c")

    def do_block(g, src, dst):
        r0 = g * WROWS
        pltpu.make_async_copy(src.at[pl.ds(r0, WROWS)], buf, sem).start()
        pltpu.make_async_copy(src.at[pl.ds(r0, WROWS)], buf, sem).wait()

        def rows16(i, carry):
            for r in range(LANES):
                for q in range(D // LANES):
                    sl = pl.ds(q * LANES, LANES)
                    wbuf[i * LANES + r, sl] = buf[i * LANES + r, sl]
            return carry

        lax.fori_loop(0, WROWS // LANES, rows16, 0)
        pltpu.make_async_copy(wbuf, dst.at[pl.ds(r0, WROWS)], sem).start()
        pltpu.make_async_copy(wbuf, dst.at[pl.ds(r0, WROWS)], sem).wait()

    def round_body(k, carry):
        g = k * NW + wid
        do_block(g, ctx_hbm, ctxw_hbm)
        do_block(g, cen_hbm, cenw_hbm)
        return carry

    lax.fori_loop(0, WPW, round_body, 0)
    rem = WBLOCKS - WPW * NW

    @pl.when(wid < rem)
    def _():
        g = WPW * NW + wid
        do_block(g, ctx_hbm, ctxw_hbm)
        do_block(g, cen_hbm, cenw_hbm)


def _cbow_sc_body(cw_hbm, neg_hbm, ctx_hbm, cen_hbm, out_hbm,
                  idx_all_c, idx_all_n, ctx_rows, neg_rows, dot_bufs,
                  sems_c, sems_n, sems_o):
    wid = lax.axis_index("s") * NC + lax.axis_index("c")
    ibase = wid * IDX_PER_W

    pltpu.sync_copy(cw_hbm.at[pl.ds(ibase, IDX_PER_W)], idx_all_c)
    pltpu.sync_copy(neg_hbm.at[pl.ds(ibase, IDX_PER_W)], idx_all_n)

    def gathers(c, p):
        descs = []
        for off, ln in STREAMS:
            src = c * IDX_PER_CHUNK + off
            descs.append(pltpu.make_async_copy(
                ctx_hbm.at[idx_all_c.at[pl.ds(src, ln)]],
                ctx_rows[p].at[pl.ds(off, ln)], sems_c[p]))
            descs.append(pltpu.make_async_copy(
                cen_hbm.at[idx_all_n.at[pl.ds(src, ln)]],
                neg_rows[p].at[pl.ds(off, ln)], sems_n[p]))
        return descs

    def out_desc(c, p):
        row0 = wid * ROWS_PER_W + c * CHUNK
        return pltpu.make_async_copy(
            dot_bufs[p], out_hbm.at[pl.ds(row0 * PART, CHUNK * PART)],
            sems_o[p])

    for p in (0, 1):
        for d in gathers(p, p):
            d.start()

    def pair_body(cc, carry):
        for p in (0, 1):
            c = 2 * cc + p
            for d in gathers(c, p):
                d.wait()

            @pl.when(c >= 2)
            def _():
                out_desc(c, p).wait()

            def row_body(i, carry2):
                base = i * L
                v = []
                for q in range(D // LANES):
                    sl = pl.ds(q * LANES, LANES)
                    acc = ctx_rows[p][base, sl]
                    for j in range(1, L):
                        acc = acc + ctx_rows[p][base + j, sl]
                    v.append(acc)
                for n in range(N):
                    sl = pl.ds(0, LANES)
                    acc = v[0] * neg_rows[p][base + n, sl]
                    for q in range(1, D // LANES):
                        sl = pl.ds(q * LANES, LANES)
                        acc = acc + v[q] * neg_rows[p][base + n, sl]
                    dot_bufs[p][pl.ds(i * PART + n * LANES, LANES)] = acc
                return carry2

            lax.fori_loop(0, CHUNK, row_body, 0)
            out_desc(c, p).start()

            @pl.when(c + 2 < NCHUNK)
            def _():
                for d in gathers(c + 2, p):
                    d.start()
        return carry

    lax.fori_loop(0, NCHUNK // 2, pair_body, 0)

    for p in (0, 1):
        out_desc(NCHUNK - 2 + p, p).wait()


def _reduce_tc_body(part_ref, mask_ref, out_ref):
    j = lax.broadcasted_iota(jnp.int32, (PART, N), 0)
    n = lax.broadcasted_iota(jnp.int32, (PART, N), 1)
    fold = jnp.where(j // LANES == n, 1.0, 0.0).astype(jnp.float32)
    red = jnp.dot(part_ref[...], fold, preferred_element_type=jnp.float32)
    out_ref[...] = red / mask_ref[...]


def kernel(center_words, context_negatives, mask_c, context_table, center_table):
    cw = center_words.astype(jnp.int32).reshape(B * L)
    neg = context_negatives.astype(jnp.int32).reshape(B * N)

    mesh = plsc.VectorSubcoreMesh(core_axis_name="c", subcore_axis_name="s")
    widen = functools.partial(
        pl.kernel,
        mesh=mesh,
        out_type=(jax.ShapeDtypeStruct((VOCAB, WIDE), jnp.float32),
                  jax.ShapeDtypeStruct((VOCAB, WIDE), jnp.float32)),
        scratch_types=[
            pltpu.VMEM((WROWS, D), jnp.float32),
            pltpu.VMEM((WROWS, WIDE), jnp.float32),
            pltpu.SemaphoreType.DMA,
        ],
        compiler_params=pltpu.CompilerParams(use_tc_tiling_on_sc=True),
    )(_widen_sc_body)
    ctx_w, cen_w = widen(context_table, center_table)

    sc_run = functools.partial(
        pl.kernel,
        mesh=mesh,
        out_type=jax.ShapeDtypeStruct((B * PART,), jnp.float32),
        scratch_types=[
            pltpu.VMEM((IDX_PER_W,), jnp.int32),
            pltpu.VMEM((IDX_PER_W,), jnp.int32),
            [pltpu.VMEM((IDX_PER_CHUNK, WIDE), jnp.float32) for _ in range(2)],
            [pltpu.VMEM((IDX_PER_CHUNK, WIDE), jnp.float32) for _ in range(2)],
            [pltpu.VMEM((CHUNK * PART,), jnp.float32) for _ in range(2)],
            [pltpu.SemaphoreType.DMA for _ in range(2)],
            [pltpu.SemaphoreType.DMA for _ in range(2)],
            [pltpu.SemaphoreType.DMA for _ in range(2)],
        ],
        compiler_params=pltpu.CompilerParams(use_tc_tiling_on_sc=True),
    )(_cbow_sc_body)
    part = sc_run(cw, neg, ctx_w, cen_w)

    out = pl.pallas_call(
        _reduce_tc_body,
        grid=(B // TC_BLOCK,),
        in_specs=[
            pl.BlockSpec((TC_BLOCK, PART), lambda i: (i, 0)),
            pl.BlockSpec((TC_BLOCK, 1), lambda i: (i, 0)),
        ],
        out_specs=pl.BlockSpec((TC_BLOCK, N), lambda i: (i, 0)),
        out_shape=jax.ShapeDtypeStruct((B, N), jnp.float32),
    )(part.reshape(B, PART), mask_c.reshape(B, 1))
    return out.reshape(B, 1, N)


# speedup vs baseline: 1.1687x; 1.1687x over previous
import functools

import jax
import jax.numpy as jnp
from jax import lax
from jax.experimental import pallas as pl
from jax.experimental.pallas import tpu as pltpu
from jax.experimental.pallas import tpu_sc as plsc

B = 16384
L = 20
N = 20
D = 64
VOCAB = 1000000
WIDE = 128
LANES = 16
NC = 2
NS = 16
NW = NC * NS

ROWS_PER_W = B // NW
CHUNK = 8
NCHUNK = ROWS_PER_W // CHUNK
IDX_PER_CHUNK = CHUNK * L
IDX_PER_W = ROWS_PER_W * L
PART = N * LANES
STREAMS = ((0, 128), (128, 32))

TC_BLOCK = 256
DEPAD_ROWS = 8000


def _depad_body(in_ref, out_ref):
    out_ref[:, 0:D] = in_ref[...]


def _depad(table):
    return pl.pallas_call(
        _depad_body,
        grid=(VOCAB // DEPAD_ROWS,),
        in_specs=[pl.BlockSpec((DEPAD_ROWS, D), lambda i: (i, 0))],
        out_specs=pl.BlockSpec((DEPAD_ROWS, WIDE), lambda i: (i, 0)),
        out_shape=jax.ShapeDtypeStruct((VOCAB, WIDE), jnp.float32),
    )(table)


def _cbow_sc_body(cw_hbm, neg_hbm, ctx_hbm, cen_hbm, out_hbm,
                  idx_all_c, idx_all_n, ctx_rows, neg_rows, dot_bufs,
                  sems_c, sems_n, sems_o):
    wid = lax.axis_index("s") * NC + lax.axis_index("c")
    ibase = wid * IDX_PER_W

    pltpu.sync_copy(cw_hbm.at[pl.ds(ibase, IDX_PER_W)], idx_all_c)
    pltpu.sync_copy(neg_hbm.at[pl.ds(ibase, IDX_PER_W)], idx_all_n)

    def gathers(c, p):
        descs = []
        for off, ln in STREAMS:
            src = c * IDX_PER_CHUNK + off
            descs.append(pltpu.make_async_copy(
                ctx_hbm.at[idx_all_c.at[pl.ds(src, ln)]],
                ctx_rows[p].at[pl.ds(off, ln)], sems_c[p]))
            descs.append(pltpu.make_async_copy(
                cen_hbm.at[idx_all_n.at[pl.ds(src, ln)]],
                neg_rows[p].at[pl.ds(off, ln)], sems_n[p]))
        return descs

    def out_desc(c, p):
        row0 = wid * ROWS_PER_W + c * CHUNK
        return pltpu.make_async_copy(
            dot_bufs[p], out_hbm.at[pl.ds(row0 * PART, CHUNK * PART)],
            sems_o[p])

    for p in (0, 1):
        for d in gathers(p, p):
            d.start()

    def pair_body(cc, carry):
        for p in (0, 1):
            c = 2 * cc + p
            for d in gathers(c, p):
                d.wait()

            @pl.when(c >= 2)
            def _():
                out_desc(c, p).wait()

            def row_body(i, carry2):
                base = i * L
                v = []
                for q in range(D // LANES):
                    sl = pl.ds(q * LANES, LANES)
                    acc = ctx_rows[p][base, sl]
                    for j in range(1, L):
                        acc = acc + ctx_rows[p][base + j, sl]
                    v.append(acc)
                for n in range(N):
                    sl = pl.ds(0, LANES)
                    acc = v[0] * neg_rows[p][base + n, sl]
                    for q in range(1, D // LANES):
                        sl = pl.ds(q * LANES, LANES)
                        acc = acc + v[q] * neg_rows[p][base + n, sl]
                    dot_bufs[p][pl.ds(i * PART + n * LANES, LANES)] = acc
                return carry2

            lax.fori_loop(0, CHUNK, row_body, 0)
            out_desc(c, p).start()

            @pl.when(c + 2 < NCHUNK)
            def _():
                for d in gathers(c + 2, p):
                    d.start()
        return carry

    lax.fori_loop(0, NCHUNK // 2, pair_body, 0)

    for p in (0, 1):
        out_desc(NCHUNK - 2 + p, p).wait()


def _reduce_tc_body(part_ref, mask_ref, out_ref):
    j = lax.broadcasted_iota(jnp.int32, (PART, N), 0)
    n = lax.broadcasted_iota(jnp.int32, (PART, N), 1)
    fold = jnp.where(j // LANES == n, 1.0, 0.0).astype(jnp.float32)
    red = jnp.dot(part_ref[...], fold, preferred_element_type=jnp.float32)
    out_ref[...] = red / mask_ref[...]


def kernel(center_words, context_negatives, mask_c, context_table, center_table):
    cw = center_words.astype(jnp.int32).reshape(B * L)
    neg = context_negatives.astype(jnp.int32).reshape(B * N)

    mesh = plsc.VectorSubcoreMesh(core_axis_name="c", subcore_axis_name="s")
    sc_run = functools.partial(
        pl.kernel,
        mesh=mesh,
        out_type=jax.ShapeDtypeStruct((B * PART,), jnp.float32),
        scratch_types=[
            pltpu.VMEM((IDX_PER_W,), jnp.int32),
            pltpu.VMEM((IDX_PER_W,), jnp.int32),
            [pltpu.VMEM((IDX_PER_CHUNK, WIDE), jnp.float32) for _ in range(2)],
            [pltpu.VMEM((IDX_PER_CHUNK, WIDE), jnp.float32) for _ in range(2)],
            [pltpu.VMEM((CHUNK * PART,), jnp.float32) for _ in range(2)],
            [pltpu.SemaphoreType.DMA for _ in range(2)],
            [pltpu.SemaphoreType.DMA for _ in range(2)],
            [pltpu.SemaphoreType.DMA for _ in range(2)],
        ],
        compiler_params=pltpu.CompilerParams(use_tc_tiling_on_sc=True),
    )(_cbow_sc_body)
    part = sc_run(cw, neg, _depad(context_table), _depad(center_table))

    out = pl.pallas_call(
        _reduce_tc_body,
        grid=(B // TC_BLOCK,),
        in_specs=[
            pl.BlockSpec((TC_BLOCK, PART), lambda i: (i, 0)),
            pl.BlockSpec((TC_BLOCK, 1), lambda i: (i, 0)),
        ],
        out_specs=pl.BlockSpec((TC_BLOCK, N), lambda i: (i, 0)),
        out_shape=jax.ShapeDtypeStruct((B, N), jnp.float32),
    )(part.reshape(B, PART), mask_c.reshape(B, 1))
    return out.reshape(B, 1, N)
